# baseline (speedup 1.0000x reference)
import jax
import jax.numpy as jnp
from jax.experimental import pallas as pl
from jax.experimental.pallas import tpu as pltpu

_F32 = jnp.float32


def _body(adj_hbm, x_ref, c_ref, w1_ref, b1_ref, wk_ref,
          chebb_ref, out_ref, adj_ref, dis_s, ysum_s, yb_s, y2s_s, u_s, sem):
    p = pl.program_id(0)
    i = pl.program_id(1)
    tr = out_ref.shape[0]
    d_dim = x_ref.shape[-1]
    rows = pl.ds(i * tr, tr)

    @pl.when((p == 0) & (i == 0))
    def _start_adj_copy():
        pltpu.make_async_copy(adj_hbm, adj_ref, sem).start()

    @pl.when(p == 0)
    def _prep():
        cv = c_ref[...]
        silu = cv * jax.nn.sigmoid(cv)
        mod = jnp.dot(silu, w1_ref[...],
                      preferred_element_type=_F32) + b1_ref[...]
        shift = mod[:, :d_dim]
        scale = mod[:, d_dim:]

        xb = x_ref[...]
        mu = jnp.mean(xb, axis=-1, keepdims=True)
        m2 = jnp.mean(xb * xb, axis=-1, keepdims=True)
        var = m2 - mu * mu
        r = 1.0 / jnp.sqrt(var + 1e-6)
        g = (1.0 + scale[:, None, :]) * r
        xm = xb * g + (shift[:, None, :] - mu * g)

        w0 = wk_ref[0, 0]
        w1k = wk_ref[1, 0]
        w2k = wk_ref[2, 0]
        wsum24 = jnp.tile(w0 + w1k + w2k, (1, 8))
        wb24 = jnp.tile(w1k + 4.0 * w2k, (1, 8))
        w224 = jnp.tile(w2k, (1, 8))
        lane_b = jax.lax.broadcasted_iota(jnp.int32, (1, 24), 1) // 3

        ps = jnp.zeros((tr, 24), _F32)
        pb = jnp.zeros((tr, 24), _F32)
        p2 = jnp.zeros((tr, 24), _F32)
        for b in range(xb.shape[0]):
            xbb = xm[b]
            m = (lane_b == b).astype(_F32)
            ps = ps + jnp.dot(xbb, wsum24 * m, preferred_element_type=_F32)
            pb = pb + jnp.dot(xbb, wb24 * m, preferred_element_type=_F32)
            p2 = p2 + jnp.dot(xbb, w224 * m, preferred_element_type=_F32)
        ysum_s[rows, :] = ps
        yb_s[rows, :] = pb
        y2s_s[rows, :] = p2

    @pl.when((p == 1) & (i == 0))
    def _wait_adj_copy():
        pltpu.make_async_copy(adj_hbm, adj_ref, sem).wait()

    @pl.when(p == 1)
    def _rowsum():
        a = adj_ref[rows, :]
        d = jnp.sum(a, axis=1, keepdims=True)
        dis = 1.0 / jnp.sqrt(d)
        dis_s[rows, :] = dis
        y2s_s[rows, :] = y2s_s[rows, :] * dis

    @pl.when(p == 2)
    def _pass1():
        u_s[rows, :] = jnp.dot(adj_ref[rows, :], y2s_s[...],
                               preferred_element_type=_F32)

    @pl.when((p == 3) & (i == 0))
    def _make_rhs():
        disf = dis_s[...]
        u_s[...] = disf * yb_s[...] - 2.0 * (disf * disf) * u_s[...]

    @pl.when(p == 3)
    def _pass2():
        v = jnp.dot(adj_ref[rows, :], u_s[...], preferred_element_type=_F32)
        out_ref[...] = ysum_s[rows, :] - dis_s[rows, :] * v + chebb_ref[...]


def kernel(x, adj, c, W1, b1, cheb_w, cheb_b):
    B, N, D = x.shape
    TR = 1024
    grid = (4, N // TR)

    c2 = c.reshape(B, D)
    b1r = b1.reshape(1, 2 * D)
    chebb24 = jnp.tile(cheb_b.reshape(1, 3), (1, B))

    full = lambda shape: pl.BlockSpec(shape,
                                      lambda p, i: tuple(0 for _ in shape))

    out24 = pl.pallas_call(
        _body,
        grid=grid,
        in_specs=[
            pl.BlockSpec(memory_space=pl.ANY),
            pl.BlockSpec((B, TR, D),
                         lambda p, i: (0, jnp.where(p == 0, i, 0), 0)),
            full((B, D)),
            full((D, 2 * D)),
            full((1, 2 * D)),
            full((3, 1, D, 3)),
            full((1, 3 * B)),
        ],
        out_specs=pl.BlockSpec((TR, 3 * B),
                               lambda p, i: (jnp.where(p == 3, i, 0), 0)),
        out_shape=jax.ShapeDtypeStruct((N, 3 * B), _F32),
        scratch_shapes=[
            pltpu.VMEM((N, N), _F32),
            pltpu.VMEM((N, 1), _F32),
            pltpu.VMEM((N, 3 * B), _F32),
            pltpu.VMEM((N, 3 * B), _F32),
            pltpu.VMEM((N, 3 * B), _F32),
            pltpu.VMEM((N, 3 * B), _F32),
            pltpu.SemaphoreType.DMA,
        ],
    )(adj, x, c2, W1, b1r, cheb_w, chebb24)

    return jnp.transpose(out24.reshape(N, B, 3), (1, 0, 2))


# speedup vs baseline: 1.0216x; 1.0216x over previous
import jax
import jax.numpy as jnp
from jax.experimental import pallas as pl
from jax.experimental.pallas import tpu as pltpu

_F32 = jnp.float32


def _body(adj_hbm, x_ref, c_ref, w1_ref, b1_ref, wk_ref,
          chebb_ref, out_ref, adj_ref, dis_s, ysum_s, yb_s, y2s_s, u_s, sem):
    p = pl.program_id(0)
    i = pl.program_id(1)
    tr = out_ref.shape[0]
    d_dim = x_ref.shape[-1]
    rows = pl.ds(i * tr, tr)

    @pl.when((p == 0) & (i == 0))
    def _start_adj_copy():
        pltpu.make_async_copy(adj_hbm, adj_ref, sem).start()

    @pl.when(p == 0)
    def _prep():
        cv = c_ref[...]
        silu = cv * jax.nn.sigmoid(cv)
        mod = jnp.dot(silu, w1_ref[...],
                      preferred_element_type=_F32) + b1_ref[...]
        shift = mod[:, :d_dim]
        scale = mod[:, d_dim:]

        xb = x_ref[...]
        mu = jnp.mean(xb, axis=-1, keepdims=True)
        m2 = jnp.mean(xb * xb, axis=-1, keepdims=True)
        var = m2 - mu * mu
        r = 1.0 / jnp.sqrt(var + 1e-6)
        g = (1.0 + scale[:, None, :]) * r
        xm = xb * g + (shift[:, None, :] - mu * g)

        ps = jnp.zeros((tr, 24), _F32)
        pb = jnp.zeros((tr, 24), _F32)
        p2 = jnp.zeros((tr, 24), _F32)
        for b in range(xb.shape[0]):
            xbb = xm[b]
            ps = ps + jnp.dot(xbb, wk_ref[0, b], preferred_element_type=_F32)
            pb = pb + jnp.dot(xbb, wk_ref[1, b], preferred_element_type=_F32)
            p2 = p2 + jnp.dot(xbb, wk_ref[2, b], preferred_element_type=_F32)
        ysum_s[rows, :] = ps
        yb_s[rows, :] = pb
        y2s_s[rows, :] = p2

    @pl.when((p == 1) & (i == 0))
    def _wait_adj_copy():
        pltpu.make_async_copy(adj_hbm, adj_ref, sem).wait()

    @pl.when(p == 1)
    def _rowsum():
        a = adj_ref[rows, :]
        d = jnp.sum(a, axis=1, keepdims=True)
        dis = 1.0 / jnp.sqrt(d)
        dis_s[rows, :] = dis
        y2s_s[rows, :] = y2s_s[rows, :] * dis

    @pl.when(p == 2)
    def _pass1():
        u_s[rows, :] = jnp.dot(adj_ref[rows, :], y2s_s[...],
                               preferred_element_type=_F32)

    @pl.when((p == 3) & (i == 0))
    def _make_rhs():
        disf = dis_s[...]
        u_s[...] = disf * yb_s[...] - 2.0 * (disf * disf) * u_s[...]

    @pl.when(p == 3)
    def _pass2():
        v = jnp.dot(adj_ref[rows, :], u_s[...], preferred_element_type=_F32)
        out_ref[...] = ysum_s[rows, :] - dis_s[rows, :] * v + chebb_ref[...]


def kernel(x, adj, c, W1, b1, cheb_w, cheb_b):
    B, N, D = x.shape
    TR = 1024
    grid = (4, N // TR)

    c2 = c.reshape(B, D)
    b1r = b1.reshape(1, 2 * D)
    wk = cheb_w.reshape(3, 1, D, 3)
    wall = jnp.concatenate([wk[0] + wk[1] + wk[2], wk[1] + 4.0 * wk[2],
                            wk[2]], axis=0)
    eyeb = jnp.eye(B, dtype=_F32).reshape(1, B, 1, B, 1)
    wcomb = (wall[:, None, :, None, :] * eyeb).reshape(3, B, D, 3 * B)
    chebb24 = jnp.tile(cheb_b.reshape(1, 3), (1, B))

    full = lambda shape: pl.BlockSpec(shape,
                                      lambda p, i: tuple(0 for _ in shape))

    out24 = pl.pallas_call(
        _body,
        grid=grid,
        in_specs=[
            pl.BlockSpec(memory_space=pl.ANY),
            pl.BlockSpec((B, TR, D),
                         lambda p, i: (0, jnp.where(p == 0, i, 0), 0)),
            full((B, D)),
            full((D, 2 * D)),
            full((1, 2 * D)),
            full((3, B, D, 3 * B)),
            full((1, 3 * B)),
        ],
        out_specs=pl.BlockSpec((TR, 3 * B),
                               lambda p, i: (jnp.where(p == 3, i, 0), 0)),
        out_shape=jax.ShapeDtypeStruct((N, 3 * B), _F32),
        scratch_shapes=[
            pltpu.VMEM((N, N), _F32),
            pltpu.VMEM((N, 1), _F32),
            pltpu.VMEM((N, 3 * B), _F32),
            pltpu.VMEM((N, 3 * B), _F32),
            pltpu.VMEM((N, 3 * B), _F32),
            pltpu.VMEM((N, 3 * B), _F32),
            pltpu.SemaphoreType.DMA,
        ],
    )(adj, x, c2, W1, b1r, wcomb, chebb24)

    return jnp.transpose(out24.reshape(N, B, 3), (1, 0, 2))
